# Initial kernel scaffold; baseline (speedup 1.0000x reference)
import jax
import jax.numpy as jnp
from jax.experimental import pallas as pl


def kernel(src_emb, rel_ids, dst_emb, rel_emb_table):
    raise NotImplementedError("write your pallas kernel here")


# speedup vs baseline: 1.4154x; 1.4154x over previous
import functools

import jax
import jax.numpy as jnp
from jax import lax
from jax.experimental import pallas as pl
from jax.experimental.pallas import tpu as pltpu
from jax.experimental.pallas import tpu_sc as plsc

B = 16384
D = 128
NUM_REL = 1000

_info = plsc.get_sparse_core_info()
NC, NS, L = _info.num_cores, _info.num_subcores, _info.num_lanes
NW = NC * NS
B_PER_W = B // NW
CHUNK = 128
N_CHUNKS = B_PER_W // CHUNK


def _sc_kernel():
    mesh = plsc.VectorSubcoreMesh(core_axis_name="c", subcore_axis_name="s")

    @functools.partial(
        pl.kernel,
        mesh=mesh,
        out_type=jax.ShapeDtypeStruct((B,), jnp.float32),
        scratch_types=[
            pltpu.VMEM((CHUNK,), jnp.int32),
            pltpu.VMEM((CHUNK, D), jnp.float32),
            pltpu.VMEM((CHUNK, D), jnp.float32),
            pltpu.VMEM((CHUNK, D), jnp.float32),
            pltpu.VMEM((CHUNK,), jnp.float32),
            pltpu.SemaphoreType.DMA,
        ],
    )
    def k(src_hbm, ids_hbm, dst_hbm, table_hbm, out_hbm,
          idx_v, rel_v, src_v, dst_v, out_v, sem):
        wid = lax.axis_index("s") * NC + lax.axis_index("c")
        base = wid * B_PER_W
        lanes = lax.iota(jnp.int32, L)

        dnums = lax.GatherDimensionNumbers(
            offset_dims=(), collapsed_slice_dims=(0,), start_index_map=(0,))

        def lane_perm(x, perm):
            return lax.gather(
                x, perm[:, None], dimension_numbers=dnums, slice_sizes=(1,),
                mode=lax.GatherScatterMode.PROMISE_IN_BOUNDS)

        def lane_sum(x):
            for k in (1, 2, 4, 8):
                x = x + lane_perm(x, jnp.bitwise_xor(lanes, k))
            return x

        for c in range(N_CHUNKS):
            rb = base + c * CHUNK
            pltpu.sync_copy(ids_hbm.at[pl.ds(rb, CHUNK)], idx_v)
            pltpu.async_copy(table_hbm.at[idx_v], rel_v, sem).wait()
            pltpu.sync_copy(src_hbm.at[pl.ds(rb, CHUNK)], src_v)
            pltpu.sync_copy(dst_hbm.at[pl.ds(rb, CHUNK)], dst_v)

            def group_body(g, _):
                def row_body(i, res):
                    r = g * L + i
                    acc = jnp.zeros((L,), jnp.float32)
                    for j in range(D // L):
                        sl = pl.ds(j * L, L)
                        acc = acc + src_v[r, sl] * rel_v[r, sl] * dst_v[r, sl]
                    return jnp.where(lanes == i, lane_sum(acc), res)

                res = lax.fori_loop(0, L, row_body,
                                    jnp.zeros((L,), jnp.float32))
                out_v[pl.ds(g * L, L)] = res
                return 0

            lax.fori_loop(0, CHUNK // L, group_body, 0)
            pltpu.sync_copy(out_v, out_hbm.at[pl.ds(rb, CHUNK)])

    return k


_scorer = _sc_kernel()


@jax.jit
def kernel(src_emb, rel_ids, dst_emb, rel_emb_table):
    ids = rel_ids.astype(jnp.int32)
    return _scorer(src_emb, ids, dst_emb, rel_emb_table)
